# baseline (speedup 1.0000x reference)
import functools

import jax
import jax.numpy as jnp
from jax import lax
from jax.experimental import pallas as pl
from jax.experimental.pallas import tpu as pltpu
from jax.experimental.pallas import tpu_sc as plsc

_K = 10
_TR = 1024
_TE = 512


def _pcall(*args, **kwargs):
    return pl.pallas_call(*args, **kwargs)


def _topk_body(x_ref, xall_ref, idx_ref, *, n, k):
    b = pl.program_id(0)
    xr = x_ref[0]
    xa = xall_ref[0]
    d2r = jnp.sum(xr * xr, axis=1, keepdims=True)
    d2a = jnp.sum(xa * xa, axis=1)
    cross = lax.dot_general(
        xr.astype(jnp.bfloat16), xa.astype(jnp.bfloat16),
        dimension_numbers=(((1,), (1,)), ((), ())),
        preferred_element_type=jnp.float32)
    dist = d2r + d2a[None, :] - 2.0 * cross
    iota = lax.broadcasted_iota(jnp.int32, dist.shape, 1)
    cols = []
    for _ in range(k):
        m = jnp.min(dist, axis=1, keepdims=True)
        am = jnp.min(jnp.where(dist == m, iota, n), axis=1, keepdims=True)
        cols.append(am)
        dist = jnp.where(iota == am, jnp.inf, dist)
    idx_ref[0] = jnp.concatenate(cols, axis=1) + b * n


def _topk(h, k):
    B, N, dp = h.shape
    return _pcall(
        functools.partial(_topk_body, n=N, k=k),
        grid=(B, N // _TR),
        in_specs=[
            pl.BlockSpec((1, _TR, dp), lambda b, t: (b, t, 0)),
            pl.BlockSpec((1, N, dp), lambda b, t: (b, 0, 0)),
        ],
        out_specs=pl.BlockSpec((1, _TR, k), lambda b, t: (b, t, 0)),
        out_shape=jax.ShapeDtypeStruct((B, N, k), jnp.int32),
    )(h, h)


def _sc_gather(idx_flat, x2d):
    E = idx_flat.shape[0]
    BN, dp = x2d.shape
    info = plsc.get_sparse_core_info()
    nc = info.num_cores
    nw = nc * info.num_subcores
    PE = E // nw
    G = 128
    while 2 * G * dp * 4 + PE * 4 > 480 * 1024:
        G //= 2
    NP = PE // (2 * G)
    mesh = plsc.VectorSubcoreMesh(core_axis_name="c", subcore_axis_name="s")

    @functools.partial(
        pl.kernel, mesh=mesh,
        out_type=jax.ShapeDtypeStruct((E, dp), jnp.float32),
        scratch_types=[
            pltpu.VMEM((PE,), jnp.int32),
            pltpu.VMEM((G, dp), jnp.float32),
            pltpu.VMEM((G, dp), jnp.float32),
            pltpu.SemaphoreType.DMA,
            pltpu.SemaphoreType.DMA,
        ],
    )
    def run(idx_hbm, x_hbm, out_hbm, idx_v, buf0, buf1, sem0, sem1):
        wid = lax.axis_index("s") * nc + lax.axis_index("c")
        base = wid * PE
        pltpu.sync_copy(idx_hbm.at[pl.ds(base, PE)], idx_v)
        pltpu.async_copy(x_hbm.at[idx_v.at[pl.ds(0, G)]], buf0, sem0)

        def pair(gp, carry):
            o0 = gp * (2 * G)
            o1 = o0 + G
            pltpu.async_copy(x_hbm.at[idx_v.at[pl.ds(o1, G)]], buf1, sem1)
            pltpu.make_async_copy(
                x_hbm.at[idx_v.at[pl.ds(o0, G)]], buf0, sem0).wait()
            pltpu.sync_copy(buf0, out_hbm.at[pl.ds(base + o0, G)])

            @pl.when(gp + 1 < NP)
            def _():
                pltpu.async_copy(
                    x_hbm.at[idx_v.at[pl.ds(o0 + 2 * G, G)]], buf0, sem0)

            pltpu.make_async_copy(
                x_hbm.at[idx_v.at[pl.ds(o1, G)]], buf1, sem1).wait()
            pltpu.sync_copy(buf1, out_hbm.at[pl.ds(base + o1, G)])
            return carry

        lax.fori_loop(0, NP, pair, 0)

    return run(idx_flat, x2d)


def _edge_body(xg_ref, x_ref, w_ref, bias_ref, out_ref, *, k, d, dout, dpo):
    xr_b = x_ref[:, :d].astype(jnp.bfloat16)
    acc = None
    for j in range(k):
        diff = (xg_ref[:, j, :d] - x_ref[:, :d]).astype(jnp.bfloat16)
        msg = jnp.concatenate([xr_b, diff], axis=1)
        e = jnp.dot(msg, w_ref[...], preferred_element_type=jnp.float32)
        e = e + bias_ref[...]
        e = jnp.maximum(e, 0.2 * e)
        acc = e if acc is None else jnp.maximum(acc, e)
    if dpo != dout:
        acc = jnp.concatenate(
            [acc, jnp.zeros((acc.shape[0], dpo - dout), jnp.float32)], axis=1)
    out_ref[...] = acc


def _edge_mlp(xg, x2d, w, bias, k, dout):
    E, dp = xg.shape
    BN = x2d.shape[0]
    d = w.shape[0] // 2
    dpo = max(dout, 128)
    out = _pcall(
        functools.partial(_edge_body, k=k, d=d, dout=dout, dpo=dpo),
        grid=(BN // _TE,),
        in_specs=[
            pl.BlockSpec((_TE, k, dp), lambda t: (t, 0, 0)),
            pl.BlockSpec((_TE, dp), lambda t: (t, 0)),
            pl.BlockSpec((2 * d, dout), lambda t: (0, 0)),
            pl.BlockSpec((1, dout), lambda t: (0, 0)),
        ],
        out_specs=pl.BlockSpec((_TE, dpo), lambda t: (t, 0)),
        out_shape=jax.ShapeDtypeStruct((BN, dpo), jnp.float32),
    )(xg.reshape(BN, k, dp), x2d, w, bias)
    return out


def _final_body(h_ref, m0_ref, m1_ref, m2_ref, c0_ref, c1_ref, c2_ref,
                out_ref):
    hm = jnp.max(h_ref[0], axis=0, keepdims=True)
    o = jnp.dot(hm.astype(jnp.bfloat16), m0_ref[...],
                preferred_element_type=jnp.float32) + c0_ref[...]
    o = jnp.dot(o.astype(jnp.bfloat16), m1_ref[...],
                preferred_element_type=jnp.float32) + c1_ref[...]
    o = jnp.dot(o.astype(jnp.bfloat16), m2_ref[...],
                preferred_element_type=jnp.float32) + c2_ref[...]
    out_ref[0] = o


def _final(h, m0, m1, m2, c0, c1, c2):
    B, N, dh = h.shape
    f0, f1, f2 = m0.shape[1], m1.shape[1], m2.shape[1]
    out = _pcall(
        _final_body,
        grid=(B,),
        in_specs=[
            pl.BlockSpec((1, N, dh), lambda b: (b, 0, 0)),
            pl.BlockSpec((dh, f0), lambda b: (0, 0)),
            pl.BlockSpec((f0, f1), lambda b: (0, 0)),
            pl.BlockSpec((f1, f2), lambda b: (0, 0)),
            pl.BlockSpec((1, f0), lambda b: (0, 0)),
            pl.BlockSpec((1, f1), lambda b: (0, 0)),
            pl.BlockSpec((1, f2), lambda b: (0, 0)),
        ],
        out_specs=pl.BlockSpec((1, 1, f2), lambda b: (b, 0, 0)),
        out_shape=jax.ShapeDtypeStruct((B, 1, f2), jnp.float32),
    )(h, m0, m1, m2, c0, c1, c2)
    return out.reshape(B, f2)


def kernel(x, W0, b0, W1, b1, W2, b2, W3, b3, W4, b4,
           M0, c0, M1, c1, M2, c2):
    B, N, d0 = x.shape
    BN = B * N
    h = jnp.pad(x, ((0, 0), (0, 0), (0, 128 - d0)))
    for W, b in [(W0, b0), (W1, b1), (W2, b2), (W3, b3), (W4, b4)]:
        dp = h.shape[-1]
        d = W.shape[1] // 2
        dout = W.shape[0]
        wf = jnp.zeros((2 * dp, dout), jnp.float32)
        wf = wf.at[:d].set(W[:, :d].T).at[dp:dp + d].set(W[:, d:].T)
        wf = wf.astype(jnp.bfloat16)
        idx = _topk(h, _K)
        idx_t = idx.reshape(BN * _K)
        xg = _sc_gather(idx_t, h.reshape(BN, dp))
        hw = _edge_mlp(xg, h.reshape(BN, dp),
                       wf, b.reshape(1, -1), _K, dout)
        h = hw.reshape(B, N, hw.shape[-1])
    return _final(h[..., :M0.shape[1]],
                  M0.T.astype(jnp.bfloat16), M1.T.astype(jnp.bfloat16),
                  M2.T.astype(jnp.bfloat16),
                  c0.reshape(1, -1), c1.reshape(1, -1), c2.reshape(1, -1))


# speedup vs baseline: 1.5810x; 1.5810x over previous
import functools

import jax
import jax.numpy as jnp
from jax import lax
from jax.experimental import pallas as pl
from jax.experimental.pallas import tpu as pltpu
from jax.experimental.pallas import tpu_sc as plsc

_K = 10
_TR = 1024
_TE = 512


def _pcall(*args, **kwargs):
    return pl.pallas_call(*args, **kwargs)


def _topk_body(x_ref, xall_ref, idx_ref, *, n, k):
    b = pl.program_id(0)
    xr = x_ref[0]
    xa = xall_ref[0]
    d2r = jnp.sum(xr * xr, axis=1, keepdims=True)
    d2a = jnp.sum(xa * xa, axis=1)
    cross = lax.dot_general(
        xr.astype(jnp.bfloat16), xa.astype(jnp.bfloat16),
        dimension_numbers=(((1,), (1,)), ((), ())),
        preferred_element_type=jnp.float32)
    dist = d2r + d2a[None, :] - 2.0 * cross
    iota = lax.broadcasted_iota(jnp.int32, dist.shape, 1)
    cols = []
    for _ in range(k):
        m = jnp.min(dist, axis=1, keepdims=True)
        am = jnp.min(jnp.where(dist == m, iota, n), axis=1, keepdims=True)
        cols.append(am)
        dist = jnp.where(iota == am, jnp.inf, dist)
    idx_ref[0] = jnp.concatenate(cols, axis=1) + b * n


def _topk(h, k):
    B, N, dp = h.shape
    return _pcall(
        functools.partial(_topk_body, n=N, k=k),
        grid=(B, N // _TR),
        in_specs=[
            pl.BlockSpec((1, _TR, dp), lambda b, t: (b, t, 0)),
            pl.BlockSpec((1, N, dp), lambda b, t: (b, 0, 0)),
        ],
        out_specs=pl.BlockSpec((1, _TR, k), lambda b, t: (b, t, 0)),
        out_shape=jax.ShapeDtypeStruct((B, N, k), jnp.int32),
    )(h, h)


def _sc_gather(idx_flat, x2d):
    E = idx_flat.shape[0]
    BN, dp = x2d.shape
    info = plsc.get_sparse_core_info()
    nc = info.num_cores
    nw = nc * info.num_subcores
    PE = E // nw
    NB = 4 if dp <= 128 else 2
    G = 128
    while NB * G * dp * 4 + PE * 4 > 480 * 1024:
        G //= 2
    NQ = PE // (NB * G)
    mesh = plsc.VectorSubcoreMesh(core_axis_name="c", subcore_axis_name="s")

    @functools.partial(
        pl.kernel, mesh=mesh,
        out_type=jax.ShapeDtypeStruct((E, dp), jnp.float32),
        scratch_types=[
            pltpu.VMEM((PE,), jnp.int32),
        ] + [pltpu.VMEM((G, dp), jnp.float32) for _ in range(NB)]
          + [pltpu.SemaphoreType.DMA for _ in range(NB)],
    )
    def run(idx_hbm, x_hbm, out_hbm, idx_v, *bufsem):
        bufs = bufsem[:NB]
        sems = bufsem[NB:]
        wid = lax.axis_index("s") * nc + lax.axis_index("c")
        base = wid * PE
        pltpu.sync_copy(idx_hbm.at[pl.ds(base, PE)], idx_v)
        for s in range(NB - 1):
            pltpu.async_copy(
                x_hbm.at[idx_v.at[pl.ds(s * G, G)]], bufs[s], sems[s])

        def ring(q, carry):
            o0 = q * (NB * G)
            pltpu.async_copy(
                x_hbm.at[idx_v.at[pl.ds(o0 + (NB - 1) * G, G)]],
                bufs[NB - 1], sems[NB - 1])
            for s in range(NB):
                o = o0 + s * G
                pltpu.make_async_copy(
                    x_hbm.at[idx_v.at[pl.ds(o, G)]], bufs[s], sems[s]).wait()
                pltpu.sync_copy(bufs[s], out_hbm.at[pl.ds(base + o, G)])
                if s < NB - 1:
                    @pl.when(q + 1 < NQ)
                    def _():
                        pltpu.async_copy(
                            x_hbm.at[idx_v.at[pl.ds(o + NB * G, G)]],
                            bufs[s], sems[s])
            return carry

        lax.fori_loop(0, NQ, ring, 0)

    return run(idx_flat, x2d)


def _edge_body(xg_ref, x_ref, w_ref, bias_ref, out_ref, *, k, d, dout, dpo):
    xr_b = x_ref[:, :d].astype(jnp.bfloat16)
    acc = None
    for j in range(k):
        diff = (xg_ref[j][:, :d] - x_ref[:, :d]).astype(jnp.bfloat16)
        msg = jnp.concatenate([xr_b, diff], axis=1)
        e = jnp.dot(msg, w_ref[...], preferred_element_type=jnp.float32)
        e = e + bias_ref[...]
        e = jnp.maximum(e, 0.2 * e)
        acc = e if acc is None else jnp.maximum(acc, e)
    if dpo != dout:
        acc = jnp.concatenate(
            [acc, jnp.zeros((acc.shape[0], dpo - dout), jnp.float32)], axis=1)
    out_ref[...] = acc


def _edge_mlp(xg, x2d, w, bias, k, dout):
    E, dp = xg.shape
    BN = x2d.shape[0]
    d = w.shape[0] // 2
    dpo = max(dout, 128)
    out = _pcall(
        functools.partial(_edge_body, k=k, d=d, dout=dout, dpo=dpo),
        grid=(BN // _TE,),
        in_specs=[
            pl.BlockSpec((k, _TE, dp), lambda t: (0, t, 0)),
            pl.BlockSpec((_TE, dp), lambda t: (t, 0)),
            pl.BlockSpec((2 * d, dout), lambda t: (0, 0)),
            pl.BlockSpec((1, dout), lambda t: (0, 0)),
        ],
        out_specs=pl.BlockSpec((_TE, dpo), lambda t: (t, 0)),
        out_shape=jax.ShapeDtypeStruct((BN, dpo), jnp.float32),
    )(xg.reshape(k, BN, dp), x2d, w, bias)
    return out


def _final_body(h_ref, m0_ref, m1_ref, m2_ref, c0_ref, c1_ref, c2_ref,
                out_ref):
    hm = jnp.max(h_ref[0], axis=0, keepdims=True)
    o = jnp.dot(hm.astype(jnp.bfloat16), m0_ref[...],
                preferred_element_type=jnp.float32) + c0_ref[...]
    o = jnp.dot(o.astype(jnp.bfloat16), m1_ref[...],
                preferred_element_type=jnp.float32) + c1_ref[...]
    o = jnp.dot(o.astype(jnp.bfloat16), m2_ref[...],
                preferred_element_type=jnp.float32) + c2_ref[...]
    out_ref[0] = o


def _final(h, m0, m1, m2, c0, c1, c2):
    B, N, dh = h.shape
    f0, f1, f2 = m0.shape[1], m1.shape[1], m2.shape[1]
    out = _pcall(
        _final_body,
        grid=(B,),
        in_specs=[
            pl.BlockSpec((1, N, dh), lambda b: (b, 0, 0)),
            pl.BlockSpec((dh, f0), lambda b: (0, 0)),
            pl.BlockSpec((f0, f1), lambda b: (0, 0)),
            pl.BlockSpec((f1, f2), lambda b: (0, 0)),
            pl.BlockSpec((1, f0), lambda b: (0, 0)),
            pl.BlockSpec((1, f1), lambda b: (0, 0)),
            pl.BlockSpec((1, f2), lambda b: (0, 0)),
        ],
        out_specs=pl.BlockSpec((1, 1, f2), lambda b: (b, 0, 0)),
        out_shape=jax.ShapeDtypeStruct((B, 1, f2), jnp.float32),
    )(h, m0, m1, m2, c0, c1, c2)
    return out.reshape(B, f2)


def kernel(x, W0, b0, W1, b1, W2, b2, W3, b3, W4, b4,
           M0, c0, M1, c1, M2, c2):
    B, N, d0 = x.shape
    BN = B * N
    h = jnp.pad(x, ((0, 0), (0, 0), (0, 128 - d0)))
    for W, b in [(W0, b0), (W1, b1), (W2, b2), (W3, b3), (W4, b4)]:
        dp = h.shape[-1]
        d = W.shape[1] // 2
        dout = W.shape[0]
        wf = jnp.zeros((2 * dp, dout), jnp.float32)
        wf = wf.at[:d].set(W[:, :d].T).at[dp:dp + d].set(W[:, d:].T)
        wf = wf.astype(jnp.bfloat16)
        idx = _topk(h, _K)
        idx_t = idx.reshape(BN, _K).T.reshape(BN * _K)
        xg = _sc_gather(idx_t, h.reshape(BN, dp))
        hw = _edge_mlp(xg, h.reshape(BN, dp),
                       wf, b.reshape(1, -1), _K, dout)
        h = hw.reshape(B, N, hw.shape[-1])
    return _final(h[..., :M0.shape[1]],
                  M0.T.astype(jnp.bfloat16), M1.T.astype(jnp.bfloat16),
                  M2.T.astype(jnp.bfloat16),
                  c0.reshape(1, -1), c1.reshape(1, -1), c2.reshape(1, -1))
